# Initial kernel scaffold; baseline (speedup 1.0000x reference)
import jax
import jax.numpy as jnp
from jax.experimental import pallas as pl


def kernel(x, knot_y):
    raise NotImplementedError("write your pallas kernel here")


# speedup vs baseline: 4948.4863x; 4948.4863x over previous
import jax
import jax.numpy as jnp
from jax.experimental import pallas as pl

_FEATURES = 2048
_K = 8
_XMIN = -3.0
_XMAX = 3.0
_DELTA = (_XMAX - _XMIN) / float(_K - 1)
_INV_DELTA = 1.0 / _DELTA


def _spline_body(x_ref, ky_ref, o_ref):
    x = x_ref[...]
    xc = jnp.clip(x, _XMIN, _XMAX)
    pos = (xc - _XMIN) * _INV_DELTA
    idx0 = jnp.minimum(pos.astype(jnp.int32), _K - 2)
    frac = pos - idx0.astype(jnp.float32)
    y0 = ky_ref[0:1, :]
    y1 = ky_ref[1:2, :]
    for k in range(1, _K - 1):
        m = idx0 >= k
        y0 = jnp.where(m, ky_ref[k:k + 1, :], y0)
        y1 = jnp.where(m, ky_ref[k + 1:k + 2, :], y1)
    o_ref[...] = y0 + frac * (y1 - y0)


def kernel(x, knot_y):
    rows = x.size // _FEATURES
    flat = x.reshape(rows, _FEATURES)
    kyT = knot_y.T
    br = 512
    out = pl.pallas_call(
        _spline_body,
        grid=(rows // br,),
        in_specs=[
            pl.BlockSpec((br, _FEATURES), lambda i: (i, 0)),
            pl.BlockSpec((_K, _FEATURES), lambda i: (0, 0)),
        ],
        out_specs=pl.BlockSpec((br, _FEATURES), lambda i: (i, 0)),
        out_shape=jax.ShapeDtypeStruct(flat.shape, flat.dtype),
    )(flat, kyT)
    return out.reshape(x.shape)
